# baseline (speedup 1.0000x reference)
import functools
import math

import jax
import jax.numpy as jnp
from jax.experimental import pallas as pl

IMG = 224
PATCH = 16
NPATCH = (IMG // PATCH) ** 2
D = 768
HID = 256
NE = 8
TOPK = 2
NH = 8
B = 32
HD = D // NH


def _mm_body(x_ref, w_ref, b_ref, o_ref):
    acc = jax.lax.dot_general(
        x_ref[...], w_ref[...],
        dimension_numbers=(((1,), (1,)), ((), ())),
        preferred_element_type=jnp.float32)
    o_ref[...] = acc + b_ref[...]


def _mm(x, W, b, bm=128):
    M, K = x.shape
    N = W.shape[0]
    grid = M // bm
    return pl.pallas_call(
        _mm_body,
        grid=(grid,),
        in_specs=[
            pl.BlockSpec((bm, K), lambda i: (i, 0)),
            pl.BlockSpec((N, K), lambda i: (0, 0)),
            pl.BlockSpec((1, N), lambda i: (0, 0)),
        ],
        out_specs=pl.BlockSpec((bm, N), lambda i: (i, 0)),
        out_shape=jax.ShapeDtypeStruct((M, N), jnp.float32),
    )(x, W, b.reshape(1, N))


_ATT_C = 8


def _att_body(q_ref, k_ref, v_ref, o_ref):
    C = _ATT_C
    L = B
    n = C * L
    q = q_ref[...].reshape(n, HD)
    k = k_ref[...].reshape(n, HD)
    v = v_ref[...].reshape(n, HD)
    s = jax.lax.dot_general(
        q, k, dimension_numbers=(((1,), (1,)), ((), ())),
        preferred_element_type=jnp.float32) * (1.0 / math.sqrt(HD))
    bi = jax.lax.broadcasted_iota(jnp.int32, (n, n), 0) // L
    bj = jax.lax.broadcasted_iota(jnp.int32, (n, n), 1) // L
    s = jnp.where(bi == bj, s, -1e30)
    m = jnp.max(s, axis=1, keepdims=True)
    e = jnp.exp(s - m)
    p = e / jnp.sum(e, axis=1, keepdims=True)
    o = jax.lax.dot_general(
        p, v, dimension_numbers=(((1,), (0,)), ((), ())),
        preferred_element_type=jnp.float32)
    o_ref[...] = o.reshape(C, L, HD)


def _attention(q, k, v):
    H = q.shape[0]
    C = _ATT_C
    spec = pl.BlockSpec((C, B, HD), lambda i: (i, 0, 0))
    return pl.pallas_call(
        _att_body,
        grid=(H // C,),
        in_specs=[spec, spec, spec],
        out_specs=spec,
        out_shape=jax.ShapeDtypeStruct((H, B, HD), jnp.float32),
    )(q, k, v)


def _moe_body(x_ref, wg_ref, bg_ref, w1_ref, b1_ref, w2_ref, b2_ref, o_ref):
    x = x_ref[...]
    logits = jax.lax.dot_general(
        x, wg_ref[...], dimension_numbers=(((1,), (1,)), ((), ())),
        preferred_element_type=jnp.float32) + bg_ref[...]
    lm = jnp.max(logits, axis=1, keepdims=True)
    ex = jnp.exp(logits - lm)
    probs = ex / jnp.sum(ex, axis=1, keepdims=True)

    w_list = []
    for e in range(NE):
        pe = probs[:, e:e + 1]
        gt = probs > pe
        if e > 0:
            eqlow = (probs == pe) & (jax.lax.broadcasted_iota(
                jnp.int32, probs.shape, 1) < e)
            gt = gt | eqlow
        rank = jnp.sum(gt.astype(jnp.float32), axis=1, keepdims=True)
        w_list.append(jnp.where(rank < TOPK, pe, 0.0))
    wsum = w_list[0]
    for e in range(1, NE):
        wsum = wsum + w_list[e]

    acc = None
    for e in range(NE):
        h = jax.lax.dot_general(
            x, w1_ref[e], dimension_numbers=(((1,), (1,)), ((), ())),
            preferred_element_type=jnp.float32) + b1_ref[e:e + 1, :]
        h = jnp.maximum(h, 0.0)
        y = jax.lax.dot_general(
            h, w2_ref[e], dimension_numbers=(((1,), (1,)), ((), ())),
            preferred_element_type=jnp.float32) + b2_ref[e:e + 1, :]
        contrib = (w_list[e] / wsum) * y
        acc = contrib if acc is None else acc + contrib
    o_ref[...] = acc


def _moe(x, p, bm=128):
    M = x.shape[0]
    return pl.pallas_call(
        _moe_body,
        grid=(M // bm,),
        in_specs=[
            pl.BlockSpec((bm, D), lambda i: (i, 0)),
            pl.BlockSpec((NE, D), lambda i: (0, 0)),
            pl.BlockSpec((1, NE), lambda i: (0, 0)),
            pl.BlockSpec((NE, HID, D), lambda i: (0, 0, 0)),
            pl.BlockSpec((NE, HID), lambda i: (0, 0)),
            pl.BlockSpec((NE, D, HID), lambda i: (0, 0, 0)),
            pl.BlockSpec((NE, D), lambda i: (0, 0)),
        ],
        out_specs=pl.BlockSpec((bm, D), lambda i: (i, 0)),
        out_shape=jax.ShapeDtypeStruct((M, D), jnp.float32),
    )(x, p['Wg'], p['bg'].reshape(1, NE), p['EW1'], p['Eb1'], p['EW2'],
      p['Eb2'])


def _moe_layer(x3, p):
    x = x3.reshape(B * NPATCH, D)
    h = _mm(x, p['Wp'], p['bp'])
    qkv = _mm(h, p['Wi'], p['bi'])
    qkv = qkv.reshape(B, NPATCH, 3, NH, HD)
    qkv = qkv.transpose(2, 1, 3, 0, 4).reshape(3, NPATCH * NH, B, HD)
    att = _attention(qkv[0], qkv[1], qkv[2])
    att = att.reshape(NPATCH, NH, B, HD).transpose(2, 0, 1, 3)
    att = att.reshape(B * NPATCH, D)
    o = _mm(att, p['Wo'], p['bo'])
    out = _moe(o, p)
    return out.reshape(B, NPATCH, D), o.reshape(B, NPATCH, D)


def kernel(x, params):
    b = x.shape[0]
    g = IMG // PATCH
    xp = x.reshape(b, 1, g, PATCH, g, PATCH).transpose(0, 1, 2, 4, 3, 5)
    xp = xp.reshape(b, 1, g * g, PATCH * PATCH).transpose(0, 2, 3, 1)
    xp = xp.reshape(b * NPATCH, PATCH * PATCH)
    xe = _mm(xp, params['Wpe'], params['bpe'])
    xe = xe.reshape(b, NPATCH, D)
    cls = jnp.broadcast_to(params['cls_token'], (b, 1, D))
    xe = jnp.concatenate([cls, xe[:, :-1, :]], axis=1) + params['pos']

    first, x1 = _moe_layer(xe, params['moe1'])
    first_flat = first.reshape(b * NPATCH, D)
    fv = _mm(first_flat, params['Wvec'], params['bvec'])
    second, x2 = _moe_layer(fv.reshape(b, NPATCH, D), params['moe2'])
    second_flat = second.reshape(b * NPATCH, D)
    sv = _mm(second_flat, params['Wvec'], params['bvec'])

    cf = _mm(first[:, 0], params['Wcls'], params['bcls'], bm=B)
    cs = _mm(second[:, 0], params['Wcls'], params['bcls'], bm=B)
    return (fv.reshape(b, NPATCH, D), sv.reshape(b, NPATCH, D), cf, cs)


# speedup vs baseline: 3.4859x; 3.4859x over previous
import math

import jax
import jax.numpy as jnp
from jax.experimental import pallas as pl

IMG = 224
PATCH = 16
G = IMG // PATCH
NPATCH = G * G
D = 768
HID = 256
NE = 8
TOPK = 2
NH = 8
B = 32
HD = D // NH
M = B * NPATCH
BM = 448


def _dot_t(x, w):
    return jax.lax.dot_general(
        x, w, dimension_numbers=(((1,), (1,)), ((), ())),
        preferred_element_type=jnp.float32)


def _dot(x, w):
    return jax.lax.dot_general(
        x, w, dimension_numbers=(((1,), (0,)), ((), ())),
        preferred_element_type=jnp.float32)


def _mm_body(x_ref, w_ref, b_ref, o_ref):
    o_ref[...] = _dot_t(x_ref[...], w_ref[...]) + b_ref[...]


def _mm(x, W, b, bm=BM):
    m, K = x.shape
    N = W.shape[0]
    return pl.pallas_call(
        _mm_body,
        grid=(m // bm,),
        in_specs=[
            pl.BlockSpec((bm, K), lambda i: (i, 0)),
            pl.BlockSpec((N, K), lambda i: (0, 0)),
            pl.BlockSpec((1, N), lambda i: (0, 0)),
        ],
        out_specs=pl.BlockSpec((bm, N), lambda i: (i, 0)),
        out_shape=jax.ShapeDtypeStruct((m, N), jnp.float32),
    )(x, W, b.reshape(1, N))


def _mmt_body(a_ref, b_ref, o_ref):
    o_ref[...] = _dot(a_ref[...], b_ref[...])


def _mm_plain(A, Bm):
    m, K = A.shape
    N = Bm.shape[1]
    return pl.pallas_call(
        _mmt_body,
        grid=(m // 128,),
        in_specs=[
            pl.BlockSpec((128, K), lambda i: (i, 0)),
            pl.BlockSpec((K, N), lambda i: (0, 0)),
        ],
        out_specs=pl.BlockSpec((128, N), lambda i: (i, 0)),
        out_shape=jax.ShapeDtypeStruct((m, N), jnp.float32),
    )(A, Bm)


def _embed_body(xs_ref, w_ref, b_ref, pos_ref, cls_ref, o_ref):
    y = _dot_t(xs_ref[...], w_ref[...]) + b_ref[...]
    npp = BM // B
    y = y.reshape(npp, B, D) + pos_ref[...].reshape(npp, 1, D)
    o_ref[...] = y.reshape(BM, D)

    @pl.when(pl.program_id(0) == 0)
    def _():
        o_ref[:B, :] = cls_ref[...]


def _embed(xs, Wpe, bpe, pos2d, clsrow):
    PP = PATCH * PATCH
    return pl.pallas_call(
        _embed_body,
        grid=(M // BM,),
        in_specs=[
            pl.BlockSpec((BM, PP), lambda i: (i, 0)),
            pl.BlockSpec((D, PP), lambda i: (0, 0)),
            pl.BlockSpec((1, D), lambda i: (0, 0)),
            pl.BlockSpec((1, BM // B, D), lambda i: (i, 0, 0)),
            pl.BlockSpec((B, D), lambda i: (0, 0)),
        ],
        out_specs=pl.BlockSpec((BM, D), lambda i: (i, 0)),
        out_shape=jax.ShapeDtypeStruct((M, D), jnp.float32),
    )(xs, Wpe, bpe.reshape(1, D),
      pos2d.reshape(M // BM, BM // B, D), clsrow)


_PC = 7


def _att_body(qkv_ref, o_ref):
    n = _PC * B
    blk = qkv_ref[...]
    bi = jax.lax.broadcasted_iota(jnp.int32, (n, n), 0) // B
    bj = jax.lax.broadcasted_iota(jnp.int32, (n, n), 1) // B
    same = bi == bj
    scale = 1.0 / math.sqrt(HD)
    for h in range(NH):
        q = blk[:, h * HD:(h + 1) * HD]
        k = blk[:, D + h * HD:D + (h + 1) * HD]
        v = blk[:, 2 * D + h * HD:2 * D + (h + 1) * HD]
        s = _dot_t(q, k) * scale
        s = jnp.where(same, s, -1e30)
        mx = jnp.max(s, axis=1, keepdims=True)
        e = jnp.exp(s - mx)
        p = e / jnp.sum(e, axis=1, keepdims=True)
        o_ref[:, h * HD:(h + 1) * HD] = _dot(p, v)


def _attention(qkv):
    n = _PC * B
    return pl.pallas_call(
        _att_body,
        grid=(M // n,),
        in_specs=[pl.BlockSpec((n, 3 * D), lambda i: (i, 0))],
        out_specs=pl.BlockSpec((n, D), lambda i: (i, 0)),
        out_shape=jax.ShapeDtypeStruct((M, D), jnp.float32),
    )(qkv)


def _moe_body(x_ref, wg_ref, bg_ref, w1_ref, b1_ref, w2_ref, b2_ref, o_ref):
    x = x_ref[...]
    logits = _dot_t(x, wg_ref[...]) + bg_ref[...]
    lm = jnp.max(logits, axis=1, keepdims=True)
    ex = jnp.exp(logits - lm)
    probs = ex / jnp.sum(ex, axis=1, keepdims=True)

    w_list = []
    for e in range(NE):
        pe = probs[:, e:e + 1]
        gt = probs > pe
        if e > 0:
            eqlow = (probs == pe) & (jax.lax.broadcasted_iota(
                jnp.int32, probs.shape, 1) < e)
            gt = gt | eqlow
        rank = jnp.sum(gt.astype(jnp.float32), axis=1, keepdims=True)
        w_list.append(jnp.where(rank < TOPK, pe, 0.0))
    w = jnp.concatenate(w_list, axis=1)
    w = w / jnp.sum(w, axis=1, keepdims=True)

    h = jnp.maximum(_dot_t(x, w1_ref[...]) + b1_ref[...], 0.0)
    ecol = jax.lax.broadcasted_iota(jnp.int32, (NE, NE * HID), 1) // HID
    erow = jax.lax.broadcasted_iota(jnp.int32, (NE, NE * HID), 0)
    expand = (ecol == erow).astype(jnp.float32)
    wex = _dot(w, expand)
    y = _dot(h * wex, w2_ref[...])
    o_ref[...] = y + _dot(w, b2_ref[...])


def _moe(x, p, W2k, bm=BM):
    return pl.pallas_call(
        _moe_body,
        grid=(M // bm,),
        in_specs=[
            pl.BlockSpec((bm, D), lambda i: (i, 0)),
            pl.BlockSpec((NE, D), lambda i: (0, 0)),
            pl.BlockSpec((1, NE), lambda i: (0, 0)),
            pl.BlockSpec((NE * HID, D), lambda i: (0, 0)),
            pl.BlockSpec((1, NE * HID), lambda i: (0, 0)),
            pl.BlockSpec((NE * HID, D), lambda i: (0, 0)),
            pl.BlockSpec((NE, D), lambda i: (0, 0)),
        ],
        out_specs=pl.BlockSpec((bm, D), lambda i: (i, 0)),
        out_shape=jax.ShapeDtypeStruct((M, D), jnp.float32),
    )(x, p['Wg'], p['bg'].reshape(1, NE), p['EW1'].reshape(NE * HID, D),
      p['Eb1'].reshape(1, NE * HID), W2k, p['Eb2'])


def _transpose8_body(x_ref, o_ref):
    o_ref[0] = x_ref[0].T


def _transpose_ew2(EW2):
    return pl.pallas_call(
        _transpose8_body,
        grid=(NE,),
        in_specs=[pl.BlockSpec((1, D, HID), lambda i: (i, 0, 0))],
        out_specs=pl.BlockSpec((1, HID, D), lambda i: (i, 0, 0)),
        out_shape=jax.ShapeDtypeStruct((NE, HID, D), jnp.float32),
    )(EW2)


def _mm_t_body(x_ref, w_ref, b_ref, ot_ref):
    y = _dot_t(x_ref[...], w_ref[...]) + b_ref[...]
    for j in range(BM // B):
        ot_ref[:, 0, j, :] = y[B * j:B * (j + 1), :]


def _mm_t2_body(x_ref, w_ref, b_ref, o_ref, ot_ref):
    y = _dot_t(x_ref[...], w_ref[...]) + b_ref[...]
    o_ref[...] = y
    for j in range(BM // B):
        ot_ref[:, 0, j, :] = y[B * j:B * (j + 1), :]


def _mm_trans(x, W, b, also_flat):
    N = W.shape[0]
    t_spec = pl.BlockSpec((B, 1, BM // B, N), lambda i: (0, i, 0, 0))
    t_shape = jax.ShapeDtypeStruct((B, M // BM, BM // B, N), jnp.float32)
    in_specs = [
        pl.BlockSpec((BM, D), lambda i: (i, 0)),
        pl.BlockSpec((N, D), lambda i: (0, 0)),
        pl.BlockSpec((1, N), lambda i: (0, 0)),
    ]
    if also_flat:
        return pl.pallas_call(
            _mm_t2_body,
            grid=(M // BM,),
            in_specs=in_specs,
            out_specs=[pl.BlockSpec((BM, N), lambda i: (i, 0)), t_spec],
            out_shape=[jax.ShapeDtypeStruct((M, N), jnp.float32), t_shape],
        )(x, W, b.reshape(1, N))
    return pl.pallas_call(
        _mm_t_body,
        grid=(M // BM,),
        in_specs=in_specs,
        out_specs=t_spec,
        out_shape=t_shape,
    )(x, W, b.reshape(1, N))


def _fold_qkv(p):
    Wq = _mm_plain(p['Wi'], p['Wp'])
    bq = _mm(p['bp'].reshape(1, D), p['Wi'], p['bi'], bm=1)
    return Wq, bq


def _moe_layer(xin, p, Wq, bq, W2k):
    qkv = _mm(xin, Wq, bq)
    att = _attention(qkv)
    o = _mm(att, p['Wo'], p['bo'])
    return _moe(o, p, W2k)


def kernel(x, params):
    xp = x.reshape(B, G, PATCH, G, PATCH).transpose(1, 3, 0, 2, 4)
    xp = xp.reshape(M, PATCH * PATCH)
    xs = jnp.concatenate(
        [jnp.zeros((B, PATCH * PATCH), jnp.float32), xp[:-B]], axis=0)
    pos2d = params['pos'][0]
    clsrow = jnp.broadcast_to(
        params['cls_token'].reshape(1, D) + pos2d[0], (B, D))

    xe = _embed(xs, params['Wpe'], params['bpe'], pos2d, clsrow)

    p1, p2 = params['moe1'], params['moe2']
    Wq1, bq1 = _fold_qkv(p1)
    Wq2, bq2 = _fold_qkv(p2)
    W2k1 = _transpose_ew2(p1['EW2']).reshape(NE * HID, D)
    W2k2 = _transpose_ew2(p2['EW2']).reshape(NE * HID, D)

    first = _moe_layer(xe, p1, Wq1, bq1, W2k1)
    fv, fv_t = _mm_trans(first, params['Wvec'], params['bvec'], True)
    fv_t = fv_t.reshape(B, NPATCH, D)
    second = _moe_layer(fv, p2, Wq2, bq2, W2k2)
    sv_t = _mm_trans(second, params['Wvec'], params['bvec'], False)
    sv_t = sv_t.reshape(B, NPATCH, D)

    cf = _mm(first[:B], params['Wcls'], params['bcls'], bm=B)
    cs = _mm(second[:B], params['Wcls'], params['bcls'], bm=B)
    return (fv_t, sv_t, cf, cs)


# speedup vs baseline: 3.6915x; 1.0590x over previous
import math

import jax
import jax.numpy as jnp
from jax.experimental import pallas as pl

IMG = 224
PATCH = 16
G = IMG // PATCH
NPATCH = G * G
D = 768
HID = 256
NE = 8
TOPK = 2
NH = 8
B = 32
HD = D // NH
M = B * NPATCH
BM = 448


def _dot_t(x, w):
    return jax.lax.dot_general(
        x, w, dimension_numbers=(((1,), (1,)), ((), ())),
        preferred_element_type=jnp.float32)


def _dot(x, w):
    return jax.lax.dot_general(
        x, w, dimension_numbers=(((1,), (0,)), ((), ())),
        preferred_element_type=jnp.float32)


def _mm_body(x_ref, w_ref, b_ref, o_ref):
    o_ref[...] = _dot_t(x_ref[...], w_ref[...]) + b_ref[...]


def _mm(x, W, b, bm=BM):
    m, K = x.shape
    N = W.shape[0]
    return pl.pallas_call(
        _mm_body,
        grid=(m // bm,),
        in_specs=[
            pl.BlockSpec((bm, K), lambda i: (i, 0)),
            pl.BlockSpec((N, K), lambda i: (0, 0)),
            pl.BlockSpec((1, N), lambda i: (0, 0)),
        ],
        out_specs=pl.BlockSpec((bm, N), lambda i: (i, 0)),
        out_shape=jax.ShapeDtypeStruct((m, N), jnp.float32),
    )(x, W, b.reshape(1, N))


def _mmt_body(a_ref, b_ref, o_ref):
    o_ref[...] = _dot(a_ref[...], b_ref[...])


def _mm_plain(A, Bm):
    m, K = A.shape
    N = Bm.shape[1]
    return pl.pallas_call(
        _mmt_body,
        grid=(m // 128,),
        in_specs=[
            pl.BlockSpec((128, K), lambda i: (i, 0)),
            pl.BlockSpec((K, N), lambda i: (0, 0)),
        ],
        out_specs=pl.BlockSpec((128, N), lambda i: (i, 0)),
        out_shape=jax.ShapeDtypeStruct((m, N), jnp.float32),
    )(A, Bm)


def _embed_body(xs_ref, w_ref, b_ref, pos_ref, cls_ref, o_ref):
    y = _dot_t(xs_ref[...], w_ref[...]) + b_ref[...]
    npp = BM // B
    y = y.reshape(npp, B, D) + pos_ref[...].reshape(npp, 1, D)
    o_ref[...] = y.reshape(BM, D)

    @pl.when(pl.program_id(0) == 0)
    def _():
        o_ref[:B, :] = cls_ref[...]


def _embed(xs, Wpe, bpe, pos2d, clsrow):
    PP = PATCH * PATCH
    return pl.pallas_call(
        _embed_body,
        grid=(M // BM,),
        in_specs=[
            pl.BlockSpec((BM, PP), lambda i: (i, 0)),
            pl.BlockSpec((D, PP), lambda i: (0, 0)),
            pl.BlockSpec((1, D), lambda i: (0, 0)),
            pl.BlockSpec((1, BM // B, D), lambda i: (i, 0, 0)),
            pl.BlockSpec((B, D), lambda i: (0, 0)),
        ],
        out_specs=pl.BlockSpec((BM, D), lambda i: (i, 0)),
        out_shape=jax.ShapeDtypeStruct((M, D), jnp.float32),
    )(xs, Wpe, bpe.reshape(1, D),
      pos2d.reshape(M // BM, BM // B, D), clsrow)


_PC = 7


def _att_body(qkv_ref, o_ref):
    n = _PC * B
    blk = qkv_ref[...]
    bi = jax.lax.broadcasted_iota(jnp.int32, (n, n), 0) // B
    bj = jax.lax.broadcasted_iota(jnp.int32, (n, n), 1) // B
    same = bi == bj
    scale = 1.0 / math.sqrt(HD)
    for h in range(NH):
        q = blk[:, h * HD:(h + 1) * HD]
        k = blk[:, D + h * HD:D + (h + 1) * HD]
        v = blk[:, 2 * D + h * HD:2 * D + (h + 1) * HD]
        s = _dot_t(q, k) * scale
        s = jnp.where(same, s, -1e30)
        mx = jnp.max(s, axis=1, keepdims=True)
        e = jnp.exp(s - mx)
        p = e / jnp.sum(e, axis=1, keepdims=True)
        o_ref[:, h * HD:(h + 1) * HD] = _dot(p, v)


def _attention(qkv):
    n = _PC * B
    return pl.pallas_call(
        _att_body,
        grid=(M // n,),
        in_specs=[pl.BlockSpec((n, 3 * D), lambda i: (i, 0))],
        out_specs=pl.BlockSpec((n, D), lambda i: (i, 0)),
        out_shape=jax.ShapeDtypeStruct((M, D), jnp.float32),
    )(qkv)


def _moe_body(x_ref, wg_ref, bg_ref, w1_ref, b1_ref, w2_ref, b2_ref, o_ref):
    x = x_ref[...]
    logits = _dot_t(x, wg_ref[...]) + bg_ref[...]
    lm = jnp.max(logits, axis=1, keepdims=True)
    ex = jnp.exp(logits - lm)
    probs = ex / jnp.sum(ex, axis=1, keepdims=True)

    w_list = []
    for e in range(NE):
        pe = probs[:, e:e + 1]
        gt = probs > pe
        if e > 0:
            eqlow = (probs == pe) & (jax.lax.broadcasted_iota(
                jnp.int32, probs.shape, 1) < e)
            gt = gt | eqlow
        rank = jnp.sum(gt.astype(jnp.float32), axis=1, keepdims=True)
        w_list.append(jnp.where(rank < TOPK, pe, 0.0))
    w = jnp.concatenate(w_list, axis=1)
    w = w / jnp.sum(w, axis=1, keepdims=True)

    h = jnp.maximum(_dot_t(x, w1_ref[...]) + b1_ref[...], 0.0)
    acc = None
    for e in range(NE):
        ye = _dot_t(h[:, e * HID:(e + 1) * HID], w2_ref[e]) + b2_ref[e:e + 1]
        contrib = w[:, e:e + 1] * ye
        acc = contrib if acc is None else acc + contrib
    o_ref[...] = acc


def _moe(x, p, bm=BM):
    return pl.pallas_call(
        _moe_body,
        grid=(M // bm,),
        in_specs=[
            pl.BlockSpec((bm, D), lambda i: (i, 0)),
            pl.BlockSpec((NE, D), lambda i: (0, 0)),
            pl.BlockSpec((1, NE), lambda i: (0, 0)),
            pl.BlockSpec((NE * HID, D), lambda i: (0, 0)),
            pl.BlockSpec((1, NE * HID), lambda i: (0, 0)),
            pl.BlockSpec((NE, D, HID), lambda i: (0, 0, 0)),
            pl.BlockSpec((NE, D), lambda i: (0, 0)),
        ],
        out_specs=pl.BlockSpec((bm, D), lambda i: (i, 0)),
        out_shape=jax.ShapeDtypeStruct((M, D), jnp.float32),
    )(x, p['Wg'], p['bg'].reshape(1, NE), p['EW1'].reshape(NE * HID, D),
      p['Eb1'].reshape(1, NE * HID), p['EW2'], p['Eb2'])


def _mm_t_body(x_ref, w_ref, b_ref, ot_ref):
    y = _dot_t(x_ref[...], w_ref[...]) + b_ref[...]
    for j in range(BM // B):
        ot_ref[:, 0, j, :] = y[B * j:B * (j + 1), :]


def _mm_t2_body(x_ref, w_ref, b_ref, o_ref, ot_ref):
    y = _dot_t(x_ref[...], w_ref[...]) + b_ref[...]
    o_ref[...] = y
    for j in range(BM // B):
        ot_ref[:, 0, j, :] = y[B * j:B * (j + 1), :]


def _mm_trans(x, W, b, also_flat):
    N = W.shape[0]
    t_spec = pl.BlockSpec((B, 1, BM // B, N), lambda i: (0, i, 0, 0))
    t_shape = jax.ShapeDtypeStruct((B, M // BM, BM // B, N), jnp.float32)
    in_specs = [
        pl.BlockSpec((BM, D), lambda i: (i, 0)),
        pl.BlockSpec((N, D), lambda i: (0, 0)),
        pl.BlockSpec((1, N), lambda i: (0, 0)),
    ]
    if also_flat:
        return pl.pallas_call(
            _mm_t2_body,
            grid=(M // BM,),
            in_specs=in_specs,
            out_specs=[pl.BlockSpec((BM, N), lambda i: (i, 0)), t_spec],
            out_shape=[jax.ShapeDtypeStruct((M, N), jnp.float32), t_shape],
        )(x, W, b.reshape(1, N))
    return pl.pallas_call(
        _mm_t_body,
        grid=(M // BM,),
        in_specs=in_specs,
        out_specs=t_spec,
        out_shape=t_shape,
    )(x, W, b.reshape(1, N))


def _moe_layer(xin, p):
    h = _mm(xin, p['Wp'], p['bp'])
    qkv = _mm(h, p['Wi'], p['bi'])
    att = _attention(qkv)
    o = _mm(att, p['Wo'], p['bo'])
    return _moe(o, p)


def kernel(x, params):
    xp = x.reshape(B, G, PATCH, G, PATCH).transpose(1, 3, 0, 2, 4)
    xp = xp.reshape(M, PATCH * PATCH)
    xs = jnp.concatenate(
        [jnp.zeros((B, PATCH * PATCH), jnp.float32), xp[:-B]], axis=0)
    pos2d = params['pos'][0]
    clsrow = jnp.broadcast_to(
        params['cls_token'].reshape(1, D) + pos2d[0], (B, D))

    xe = _embed(xs, params['Wpe'], params['bpe'], pos2d, clsrow)

    p1, p2 = params['moe1'], params['moe2']
    first = _moe_layer(xe, p1)
    fv, fv_t = _mm_trans(first, params['Wvec'], params['bvec'], True)
    fv_t = fv_t.reshape(B, NPATCH, D)
    second = _moe_layer(fv, p2)
    sv_t = _mm_trans(second, params['Wvec'], params['bvec'], False)
    sv_t = sv_t.reshape(B, NPATCH, D)

    cf = _mm(first[:B], params['Wcls'], params['bcls'], bm=B)
    cs = _mm(second[:B], params['Wcls'], params['bcls'], bm=B)
    return (fv_t, sv_t, cf, cs)
